# baseline (speedup 1.0000x reference)
import jax
import jax.numpy as jnp
from jax.experimental import pallas as pl
from jax.experimental.pallas import tpu as pltpu


def _ln_relu_centered(d, g, b, eps=1e-5):
    v = jnp.mean(d * d, axis=-1, keepdims=True)
    return jnp.maximum(d * jax.lax.rsqrt(v + eps) * g + b, 0.0)


def _decoder_kernel(x_ref, adj_hbm, wc0_ref, bc0_ref, wm0_ref, bm0_ref,
                    g0_ref, be0_ref, wc1_ref, bc1_ref, wm1_ref, bm1_ref,
                    g1_ref, be1_ref, wl_ref, bl_ref, out_ref, a_sc,
                    sem0, sem1):
    f32 = jnp.float32
    gpb, n, hdim = x_ref.shape
    half = gpb // 2
    cp0 = pltpu.make_async_copy(adj_hbm.at[pl.ds(0, half)],
                                a_sc.at[pl.ds(0, half)], sem0)
    cp1 = pltpu.make_async_copy(adj_hbm.at[pl.ds(half, half)],
                                a_sc.at[pl.ds(half, half)], sem1)
    cp0.start()
    cp1.start()

    x = x_ref[...].reshape(gpb * n, hdim)
    h_pre = jax.lax.dot_general(x, wc0_ref[...], (((1,), (1,)), ((), ())),
                                preferred_element_type=f32)

    cp0.wait()
    a0 = jnp.maximum(a_sc[pl.ds(0, half)], 0.0)
    deg0 = jnp.sum(a0, axis=1)
    cp1.wait()
    a1 = jnp.maximum(a_sc[pl.ds(half, half)], 0.0)
    deg1 = jnp.sum(a1, axis=1)
    a = jnp.concatenate([a0, a1], axis=0)
    deg = jnp.concatenate([deg0, deg1], axis=0) + 1.0
    dis = jax.lax.rsqrt(deg)[:, :, None]

    layers = ((wc0_ref, bc0_ref, wm0_ref, bm0_ref, g0_ref, be0_ref),
              (wc1_ref, bc1_ref, wm1_ref, bm1_ref, g1_ref, be1_ref))
    for li, (wc, bc, wm, bm, g, be) in enumerate(layers):
        if li == 0:
            h = h_pre
        else:
            h = jax.lax.dot_general(x, wc[...], (((1,), (1,)), ((), ())),
                                    preferred_element_type=f32)
        hs = h.reshape(gpb, n, hdim) * dis
        t = jax.lax.dot_general(a, hs, (((1,), (1,)), ((0,), (0,))),
                                preferred_element_type=f32)
        xg = (t + hs) * dis + bc[...]
        x = xg.reshape(gpb * n, hdim)
        wmv = wm[...]
        wmc = wmv - jnp.mean(wmv, axis=0, keepdims=True)
        bmv = bm[...]
        bmc = bmv - jnp.mean(bmv)
        d = jax.lax.dot_general(x, wmc, (((1,), (1,)), ((), ())),
                                preferred_element_type=f32) + bmc
        x = _ln_relu_centered(d, g[...], be[...])

    mu = jax.lax.dot_general(x, wl_ref[...], (((1,), (1,)), ((), ())),
                             preferred_element_type=f32) + bl_ref[...]
    out_ref[...] = mu.reshape(gpb, n, -1)


def kernel(node_feat, adj, W_conv0, b_conv0, W_mlp0, b_mlp0, g_ln0, beta_ln0,
           W_conv1, b_conv1, W_mlp1, b_mlp1, g_ln1, beta_ln1, W_lin, b_lin):
    g, n, h = node_feat.shape
    o = W_lin.shape[0]

    def vec(v):
        return v.reshape(1, -1)

    weights = (W_conv0, vec(b_conv0), W_mlp0, vec(b_mlp0), vec(g_ln0),
               vec(beta_ln0), W_conv1, vec(b_conv1), W_mlp1, vec(b_mlp1),
               vec(g_ln1), vec(beta_ln1), W_lin, vec(b_lin))

    def wspec(w):
        return pl.BlockSpec(w.shape, lambda i: (0,) * w.ndim)

    in_specs = [
            pl.BlockSpec((g, n, h), lambda i: (0, 0, 0)),
            pl.BlockSpec(memory_space=pltpu.MemorySpace.HBM),
        ] + [wspec(w) for w in weights]

    return pl.pallas_call(
        _decoder_kernel,
        grid=(1,),
        in_specs=in_specs,
        out_specs=pl.BlockSpec((g, n, o), lambda i: (0, 0, 0)),
        out_shape=jax.ShapeDtypeStruct((g, n, o), jnp.float32),
        scratch_shapes=[
            pltpu.VMEM((g, n, n), jnp.float32),
            pltpu.SemaphoreType.DMA,
            pltpu.SemaphoreType.DMA,
        ],
    )(node_feat, adj, *weights)


# speedup vs baseline: 1.1255x; 1.1255x over previous
import jax
import jax.numpy as jnp
from jax.experimental import pallas as pl

_GPB = 8


def _ln_relu_centered(d, g, b, eps=1e-5):
    v = jnp.mean(d * d, axis=-1, keepdims=True)
    return jnp.maximum(d * jax.lax.rsqrt(v + eps) * g + b, 0.0)


def _decoder_kernel(x_ref, adj_ref, wc0_ref, bc0_ref, wm0_ref, bm0_ref,
                    g0_ref, be0_ref, wc1_ref, bc1_ref, wm1_ref, bm1_ref,
                    g1_ref, be1_ref, wl_ref, bl_ref, out_ref):
    f32 = jnp.float32
    gpb, n, hdim = x_ref.shape
    a = jnp.maximum(adj_ref[...], 0.0)
    deg = jnp.sum(a, axis=1) + 1.0
    dis = jax.lax.rsqrt(deg)[:, :, None]
    x = x_ref[...].reshape(gpb * n, hdim)

    layers = ((wc0_ref, bc0_ref, wm0_ref, bm0_ref, g0_ref, be0_ref),
              (wc1_ref, bc1_ref, wm1_ref, bm1_ref, g1_ref, be1_ref))
    for wc, bc, wm, bm, g, be in layers:
        wmv = wm[...]
        wmc = wmv - jnp.mean(wmv, axis=0, keepdims=True)
        bmv = bm[...]
        bmc = bmv - jnp.mean(bmv)
        h = jax.lax.dot_general(x, wc[...], (((1,), (1,)), ((), ())),
                                preferred_element_type=f32)
        hs = h.reshape(gpb, n, hdim) * dis
        t = jax.lax.dot_general(a, hs, (((1,), (1,)), ((0,), (0,))),
                                preferred_element_type=f32)
        xg = (t + hs) * dis + bc[...]
        x = xg.reshape(gpb * n, hdim)
        d = jax.lax.dot_general(x, wmc, (((1,), (1,)), ((), ())),
                                preferred_element_type=f32) + bmc
        x = _ln_relu_centered(d, g[...], be[...])

    mu = jax.lax.dot_general(x, wl_ref[...], (((1,), (1,)), ((), ())),
                             preferred_element_type=f32) + bl_ref[...]
    out_ref[...] = mu.reshape(gpb, n, -1)


def kernel(node_feat, adj, W_conv0, b_conv0, W_mlp0, b_mlp0, g_ln0, beta_ln0,
           W_conv1, b_conv1, W_mlp1, b_mlp1, g_ln1, beta_ln1, W_lin, b_lin):
    g, n, h = node_feat.shape
    o = W_lin.shape[0]
    gpb = _GPB

    def vec(v):
        return v.reshape(1, -1)

    weights = (W_conv0, vec(b_conv0), W_mlp0, vec(b_mlp0), vec(g_ln0),
               vec(beta_ln0), W_conv1, vec(b_conv1), W_mlp1, vec(b_mlp1),
               vec(g_ln1), vec(beta_ln1), W_lin, vec(b_lin))

    def wspec(w):
        return pl.BlockSpec(w.shape, lambda i: (0,) * w.ndim)

    grid_spec = pl.GridSpec(
        grid=(g // gpb,),
        in_specs=[
            pl.BlockSpec((gpb, n, h), lambda i: (i, 0, 0)),
            pl.BlockSpec((gpb, n, n), lambda i: (i, 0, 0)),
        ] + [wspec(w) for w in weights],
        out_specs=pl.BlockSpec((gpb, n, o), lambda i: (i, 0, 0)),
    )

    return pl.pallas_call(
        _decoder_kernel,
        grid_spec=grid_spec,
        out_shape=jax.ShapeDtypeStruct((g, n, o), jnp.float32),
    )(node_feat, adj, *weights)
